# Initial kernel scaffold; baseline (speedup 1.0000x reference)
import jax
import jax.numpy as jnp
from jax.experimental import pallas as pl


def kernel(x, edge_index, edge_attr, u, batch, e_w1, e_b1, e_w2, e_b2, e_aw, e_ab, n1_w1, n1_b1, n1_w2, n1_b2, n_aw, n_ab, n2_w1, n2_b1, n2_w2, n2_b2, g_w1, g_b1, g_w2, g_b2, g_aw, g_ab):
    raise NotImplementedError("write your pallas kernel here")


# speedup vs baseline: 14.3641x; 14.3641x over previous
import functools

import jax
import jax.numpy as jnp
from jax import lax
from jax.experimental import pallas as pl
from jax.experimental.pallas import tpu as pltpu
from jax.experimental.pallas import tpu_sc as plsc

N = 10000
E = 320000
ND = 128
ED = 16
GD = 16
HD = 16
NB = 8

NC = 2
NS = 16
NW = NC * NS
EPW = E // NW
CH = 128
NFULL = EPW // CH
TAIL = EPW - NFULL * CH
NPT = N // NS

def _sds(shape, dtype=jnp.float32):
    return jax.ShapeDtypeStruct(shape, dtype)


def _pq_body(x_ref, w_ref, p_ref, q_ref):
    pq = jnp.dot(x_ref[...], w_ref[...], preferred_element_type=jnp.float32)
    p_ref[...] = pq[:, :HD]
    q_ref[...] = pq[:, HD:]


def _tc_pq(x, w_sd):
    return pl.pallas_call(
        _pq_body,
        out_shape=(_sds((N, HD)), _sds((N, HD))),
    )(x, w_sd)


def _make_sc_gather(mesh):
    @functools.partial(
        pl.kernel,
        out_type=_sds((E, HD)),
        mesh=mesh,
        compiler_params=pltpu.CompilerParams(use_tc_tiling_on_sc=False),
        scratch_types=[
            pltpu.VMEM((EPW,), jnp.int32),
            pltpu.VMEM((EPW,), jnp.int32),
            pltpu.VMEM((CH, HD), jnp.float32),
            pltpu.VMEM((CH, HD), jnp.float32),
            pltpu.VMEM((CH, HD), jnp.float32),
            pltpu.SemaphoreType.DMA,
            pltpu.SemaphoreType.DMA,
        ],
    )
    def sc_gather(p_hbm, q_hbm, row_hbm, col_hbm, s_hbm,
                  ridx, cidx, pbuf, qbuf, sbuf, sem1, sem2):
        wid = lax.axis_index("s") * NC + lax.axis_index("c")
        base0 = wid * EPW
        pltpu.sync_copy(row_hbm.at[pl.ds(base0, EPW)], ridx)
        pltpu.sync_copy(col_hbm.at[pl.ds(base0, EPW)], cidx)

        def chunk(j, n):
            off = j * CH
            cp1 = pltpu.async_copy(
                p_hbm.at[ridx.at[pl.ds(off, n)]], pbuf.at[pl.ds(0, n)], sem1)
            cp2 = pltpu.async_copy(
                q_hbm.at[cidx.at[pl.ds(off, n)]], qbuf.at[pl.ds(0, n)], sem2)
            cp1.wait()
            cp2.wait()

            def rowi(i, _):
                sbuf[i, :] = pbuf[i, :] + qbuf[i, :]
                return 0
            lax.fori_loop(0, n, rowi, 0, unroll=4)
            pltpu.sync_copy(sbuf.at[pl.ds(0, n)],
                            s_hbm.at[pl.ds(base0 + off, n)])

        def full(j, _):
            chunk(j, CH)
            return 0
        lax.fori_loop(0, NFULL, full, 0)
        chunk(NFULL, TAIL)

    return sc_gather


def _edge_body(s_ref, ea_ref, web_ref, w2b_ref, b1_ref, b2_ref, awb_ref,
               ab_ref, exp_ref, out_ref):
    pre = (s_ref[...]
           + jnp.dot(ea_ref[...], web_ref[...],
                     preferred_element_type=jnp.float32)
           + b1_ref[...])
    h = (jnp.dot(jax.nn.relu(pre), w2b_ref[...],
                 preferred_element_type=jnp.float32) + b2_ref[...])
    logit = (jnp.dot(h, awb_ref[...], preferred_element_type=jnp.float32)
             + ab_ref[...])
    a = jax.nn.sigmoid(logit)
    out_ref[...] = h * jnp.dot(a, exp_ref[...],
                               preferred_element_type=jnp.float32)


def _tc_edge(s8, ea8, web, w2b, b1t, b2t, awb, abt, exp8):
    er = E // 8
    br = 4000
    grid = er // br
    return pl.pallas_call(
        _edge_body,
        grid=(grid,),
        in_specs=[
            pl.BlockSpec((br, ND), lambda i: (i, 0)),
            pl.BlockSpec((br, ND), lambda i: (i, 0)),
            pl.BlockSpec((ND, ND), lambda i: (0, 0)),
            pl.BlockSpec((ND, ND), lambda i: (0, 0)),
            pl.BlockSpec((1, ND), lambda i: (0, 0)),
            pl.BlockSpec((1, ND), lambda i: (0, 0)),
            pl.BlockSpec((ND, NB), lambda i: (0, 0)),
            pl.BlockSpec((1, 1), lambda i: (0, 0)),
            pl.BlockSpec((NB, ND), lambda i: (0, 0)),
        ],
        out_specs=pl.BlockSpec((br, ND), lambda i: (i, 0)),
        out_shape=_sds((er, ND)),
    )(s8, ea8, web, w2b, b1t, b2t, awb, abt, exp8)


def _make_sc_scatter(mesh):
    @functools.partial(
        pl.kernel,
        out_type=(_sds((N, HD)), _sds((N, HD)), _sds((N,)), _sds((N,))),
        mesh=mesh,
        compiler_params=pltpu.CompilerParams(use_tc_tiling_on_sc=False),
        scratch_types=[
            pltpu.VMEM((CH,), jnp.int32),
            pltpu.VMEM((TAIL,), jnp.int32),
            pltpu.VMEM((CH, HD), jnp.float32),
            pltpu.VMEM((CH,), jnp.float32),
            pltpu.VMEM((NPT, HD), jnp.float32),
            pltpu.VMEM((N,), jnp.float32),
            pltpu.VMEM_SHARED((N, HD), jnp.float32),
            pltpu.VMEM_SHARED((N,), jnp.float32),
        ],
    )
    def sc_scatter(row_hbm, eout_hbm, sums0, sums1, cnt0, cnt1,
                   cidx, cidxt, ebuf, ones, zbuf, zcnt, acc_sum, acc_cnt):
        c = lax.axis_index("c")
        s = lax.axis_index("s")
        wid = s * NC + c
        base0 = wid * EPW

        def zrow(i, _):
            zbuf[i, :] = jnp.zeros((HD,), jnp.float32)
            return 0
        lax.fori_loop(0, NPT, zrow, 0, unroll=4)

        def onesv(k, _):
            ones[pl.ds(k * HD, HD)] = jnp.ones((HD,), jnp.float32)
            return 0
        lax.fori_loop(0, CH // HD, onesv, 0)

        def zc(k, _):
            zcnt[pl.ds(k * HD, HD)] = jnp.zeros((HD,), jnp.float32)
            return 0
        lax.fori_loop(0, N // HD, zc, 0, unroll=4)

        pltpu.sync_copy(zbuf, acc_sum.at[pl.ds(s * NPT, NPT)])

        @pl.when(s == 0)
        def _():
            pltpu.sync_copy(zcnt, acc_cnt)

        plsc.subcore_barrier()

        def full(j, _):
            off = base0 + j * CH
            pltpu.sync_copy(row_hbm.at[pl.ds(off, CH)], cidx)
            pltpu.sync_copy(eout_hbm.at[pl.ds(off, CH)], ebuf)
            pltpu.sync_copy(ebuf, acc_sum.at[cidx], add=True)
            pltpu.sync_copy(ones, acc_cnt.at[cidx], add=True)
            return 0
        lax.fori_loop(0, NFULL, full, 0)

        offt = base0 + NFULL * CH
        pltpu.sync_copy(row_hbm.at[pl.ds(offt, TAIL)], cidxt)
        pltpu.sync_copy(eout_hbm.at[pl.ds(offt, TAIL)],
                        ebuf.at[pl.ds(0, TAIL)])
        pltpu.sync_copy(ebuf.at[pl.ds(0, TAIL)], acc_sum.at[cidxt], add=True)
        pltpu.sync_copy(ones.at[pl.ds(0, TAIL)], acc_cnt.at[cidxt], add=True)

        plsc.subcore_barrier()

        @pl.when(c == 0)
        def _():
            pltpu.sync_copy(acc_sum.at[pl.ds(s * NPT, NPT)],
                            sums0.at[pl.ds(s * NPT, NPT)])

            @pl.when(s == 0)
            def _():
                pltpu.sync_copy(acc_cnt, cnt0)

        @pl.when(c == 1)
        def _():
            pltpu.sync_copy(acc_sum.at[pl.ds(s * NPT, NPT)],
                            sums1.at[pl.ds(s * NPT, NPT)])

            @pl.when(s == 0)
            def _():
                pltpu.sync_copy(acc_cnt, cnt1)

    return sc_scatter


_sc_cache = []


def _get_sc_kernels():
    if not _sc_cache:
        mesh = plsc.VectorSubcoreMesh(
            core_axis_name="c", subcore_axis_name="s",
            num_cores=NC, num_subcores=NS)
        _sc_cache.append((_make_sc_gather(mesh), _make_sc_scatter(mesh)))
    return _sc_cache[0]


def _node_body(x_ref, ps0_ref, ps1_ref, c0_ref, c1_ref, bat_ref,
               w1a_ref, w1b_ref, b1_ref, w12_ref, b12_ref, naw_ref, nab_ref,
               w2a_ref, w2b_ref, b2_ref, w22_ref, b22_ref,
               xo_ref, nms_ref, nmc_ref, ems_ref, emc_ref):
    i = pl.program_id(0)
    x = x_ref[...]
    psum = ps0_ref[...] + ps1_ref[...]
    craw = c0_ref[...] + c1_ref[...]
    agg = psum / jnp.maximum(craw, 1.0)
    pre1 = (jnp.dot(x, w1a_ref[...], preferred_element_type=jnp.float32)
            + jnp.dot(agg, w1b_ref[...], preferred_element_type=jnp.float32)
            + b1_ref[...])
    h1 = (jnp.dot(jax.nn.relu(pre1), w12_ref[...],
                  preferred_element_type=jnp.float32) + b12_ref[...])
    awg = jax.nn.sigmoid(
        jnp.dot(h1, naw_ref[...], preferred_element_type=jnp.float32)
        + nab_ref[...])
    h1 = h1 * awg
    pre2 = (jnp.dot(x, w2a_ref[...], preferred_element_type=jnp.float32)
            + jnp.dot(h1, w2b_ref[...], preferred_element_type=jnp.float32)
            + b2_ref[...])
    xo = (jnp.dot(jax.nn.relu(pre2), w22_ref[...],
                  preferred_element_type=jnp.float32) + b22_ref[...])
    xo_ref[...] = xo

    oh = (bat_ref[...] ==
          lax.broadcasted_iota(jnp.int32, (1, NB), 1).astype(jnp.float32)
          ).astype(jnp.float32)
    dn = (((0,), (0,)), ((), ()))

    @pl.when(i == 0)
    def _():
        nms_ref[...] = jnp.zeros_like(nms_ref)
        nmc_ref[...] = jnp.zeros_like(nmc_ref)
        ems_ref[...] = jnp.zeros_like(ems_ref)
        emc_ref[...] = jnp.zeros_like(emc_ref)

    nms_ref[...] += lax.dot_general(oh, xo, dn,
                                    preferred_element_type=jnp.float32)
    nmc_ref[...] += lax.dot_general(oh, jnp.ones_like(xo), dn,
                                    preferred_element_type=jnp.float32)
    ems_ref[...] += lax.dot_general(oh, psum, dn,
                                    preferred_element_type=jnp.float32)
    emc_ref[...] += lax.dot_general(
        oh, jnp.broadcast_to(craw, (craw.shape[0], HD)), dn,
        preferred_element_type=jnp.float32)


def _tc_node(x, ps0, ps1, c0, c1, batf,
             w1a, w1b, b1, w12, b12, naw, nab, w2a, w2b, b2, w22, b22):
    bn = 1000
    grid = N // bn
    return pl.pallas_call(
        _node_body,
        grid=(grid,),
        in_specs=[
            pl.BlockSpec((bn, ND), lambda i: (i, 0)),
            pl.BlockSpec((bn, HD), lambda i: (i, 0)),
            pl.BlockSpec((bn, HD), lambda i: (i, 0)),
            pl.BlockSpec((bn, 1), lambda i: (i, 0)),
            pl.BlockSpec((bn, 1), lambda i: (i, 0)),
            pl.BlockSpec((bn, 1), lambda i: (i, 0)),
            pl.BlockSpec((ND, HD), lambda i: (0, 0)),
            pl.BlockSpec((HD, HD), lambda i: (0, 0)),
            pl.BlockSpec((1, HD), lambda i: (0, 0)),
            pl.BlockSpec((HD, HD), lambda i: (0, 0)),
            pl.BlockSpec((1, HD), lambda i: (0, 0)),
            pl.BlockSpec((HD, 1), lambda i: (0, 0)),
            pl.BlockSpec((1, 1), lambda i: (0, 0)),
            pl.BlockSpec((ND, HD), lambda i: (0, 0)),
            pl.BlockSpec((HD, HD), lambda i: (0, 0)),
            pl.BlockSpec((1, HD), lambda i: (0, 0)),
            pl.BlockSpec((HD, ND), lambda i: (0, 0)),
            pl.BlockSpec((1, ND), lambda i: (0, 0)),
        ],
        out_specs=(
            pl.BlockSpec((bn, ND), lambda i: (i, 0)),
            pl.BlockSpec((NB, ND), lambda i: (0, 0)),
            pl.BlockSpec((NB, ND), lambda i: (0, 0)),
            pl.BlockSpec((NB, HD), lambda i: (0, 0)),
            pl.BlockSpec((NB, HD), lambda i: (0, 0)),
        ),
        out_shape=(
            _sds((N, ND)), _sds((NB, ND)), _sds((NB, ND)),
            _sds((NB, HD)), _sds((NB, HD)),
        ),
    )(x, ps0, ps1, c0, c1, batf,
      w1a, w1b, b1, w12, b12, naw, nab, w2a, w2b, b2, w22, b22)


def _global_body(u_ref, nms_ref, nmc_ref, ems_ref, emc_ref,
                 gu_ref, gn_ref, ge_ref, gb1_ref, gw2_ref, gb2_ref,
                 gaw_ref, gab_ref, uo_ref):
    nm = nms_ref[...] / jnp.maximum(nmc_ref[...], 1.0)
    em = ems_ref[...] / jnp.maximum(emc_ref[...], 1.0)
    pre = (jnp.dot(u_ref[...], gu_ref[...],
                   preferred_element_type=jnp.float32)
           + jnp.dot(nm, gn_ref[...], preferred_element_type=jnp.float32)
           + jnp.dot(em, ge_ref[...], preferred_element_type=jnp.float32)
           + gb1_ref[...])
    g = (jnp.dot(jax.nn.relu(pre), gw2_ref[...],
                 preferred_element_type=jnp.float32) + gb2_ref[...])
    ga = jax.nn.sigmoid(
        jnp.dot(g, gaw_ref[...], preferred_element_type=jnp.float32)
        + gab_ref[...])
    uo_ref[...] = g * ga


def _tc_global(u, nms, nmc, ems, emc, gu, gn, ge, gb1, gw2, gb2, gaw, gab):
    return pl.pallas_call(
        _global_body,
        out_shape=_sds((NB, GD)),
    )(u, nms, nmc, ems, emc, gu, gn, ge, gb1, gw2, gb2, gaw, gab)


def kernel(x, edge_index, edge_attr, u, batch,
           e_w1, e_b1, e_w2, e_b2, e_aw, e_ab,
           n1_w1, n1_b1, n1_w2, n1_b2, n_aw, n_ab,
           n2_w1, n2_b1, n2_w2, n2_b2,
           g_w1, g_b1, g_w2, g_b2, g_aw, g_ab):
    f32 = jnp.float32
    row = edge_index[0]
    col = edge_index[1]

    w_sd = jnp.concatenate([e_w1[:ND], e_w1[ND:2 * ND]], axis=1)
    we = e_w1[2 * ND:]
    eye8 = jnp.eye(NB, dtype=f32)
    web = jnp.kron(eye8, we)
    w2b = jnp.kron(eye8, e_w2)
    awb = jnp.kron(eye8, e_aw)
    exp8 = jnp.kron(eye8, jnp.ones((1, HD), f32))
    b1t = jnp.tile(e_b1, NB).reshape(1, ND)
    b2t = jnp.tile(e_b2, NB).reshape(1, ND)
    abt = e_ab.reshape(1, 1)

    sc_gather, sc_scatter = _get_sc_kernels()

    p, q = _tc_pq(x, w_sd)

    s = sc_gather(p, q, row, col)

    s8 = s.reshape(E // 8, ND)
    ea8 = edge_attr.reshape(E // 8, ND)
    eout8 = _tc_edge(s8, ea8, web, w2b, b1t, b2t, awb, abt, exp8)
    e_out = eout8.reshape(E, HD)

    sums0, sums1, cnt0, cnt1 = sc_scatter(row, e_out)

    batf = batch.astype(f32).reshape(N, 1)
    x_out, nms, nmc, ems, emc = _tc_node(
        x, sums0, sums1, cnt0.reshape(N, 1), cnt1.reshape(N, 1), batf,
        n1_w1[:ND], n1_w1[ND:], n1_b1.reshape(1, HD), n1_w2,
        n1_b2.reshape(1, HD), n_aw, n_ab.reshape(1, 1),
        n2_w1[:ND], n2_w1[ND:], n2_b1.reshape(1, HD), n2_w2,
        n2_b2.reshape(1, ND))

    u_out = _tc_global(
        u, nms, nmc, ems, emc,
        g_w1[:GD], g_w1[GD:GD + ND], g_w1[GD + ND:], g_b1.reshape(1, HD),
        g_w2, g_b2.reshape(1, GD), g_aw, g_ab.reshape(1, 1))

    return (x_out, e_out, u_out)


# speedup vs baseline: 17.0001x; 1.1835x over previous
import functools

import jax
import jax.numpy as jnp
from jax import lax
from jax.experimental import pallas as pl
from jax.experimental.pallas import tpu as pltpu
from jax.experimental.pallas import tpu_sc as plsc

N = 10000
E = 320000
ND = 128
ED = 16
GD = 16
HD = 16
NB = 8

NC = 2
NS = 16
NW = NC * NS
EPW = E // NW
NPT = N // NS

G_CH = 80
G_NCH = EPW // G_CH
G_D = 5
G_GRP = G_NCH // G_D

S_CH = 40
S_NCH = EPW // S_CH
S_R = 10
S_A = 5
S_GRP = S_NCH // S_R

def _sds(shape, dtype=jnp.float32):
    return jax.ShapeDtypeStruct(shape, dtype)


def _pq_body(x_ref, w_ref, p_ref, q_ref):
    pq = jnp.dot(x_ref[...], w_ref[...], preferred_element_type=jnp.float32)
    p_ref[...] = pq[:, :HD]
    q_ref[...] = pq[:, HD:]


def _tc_pq(x, w_sd):
    return pl.pallas_call(
        _pq_body,
        out_shape=(_sds((N, HD)), _sds((N, HD))),
    )(x, w_sd)


def _make_sc_gather(mesh):
    @functools.partial(
        pl.kernel,
        out_type=_sds((E, HD)),
        mesh=mesh,
        compiler_params=pltpu.CompilerParams(use_tc_tiling_on_sc=False),
        scratch_types=(
            [pltpu.VMEM((EPW,), jnp.int32),
             pltpu.VMEM((EPW,), jnp.int32),
             pltpu.VMEM((G_D, G_CH, HD), jnp.float32),
             pltpu.VMEM((G_D, G_CH, HD), jnp.float32),
             pltpu.VMEM((G_D, G_CH, HD), jnp.float32)]
            + [pltpu.SemaphoreType.DMA] * (2 * G_D)
        ),
    )
    def sc_gather(p_hbm, q_hbm, row_hbm, col_hbm, s_hbm,
                  ridx, cidx, pbuf, qbuf, sbuf, *sems):
        gsem = sems[:G_D]
        wsem = sems[G_D:]
        wid = lax.axis_index("s") * NC + lax.axis_index("c")
        base0 = wid * EPW
        pltpu.sync_copy(row_hbm.at[pl.ds(base0, EPW)], ridx)
        pltpu.sync_copy(col_hbm.at[pl.ds(base0, EPW)], cidx)

        def fire_gather(j, b):
            off = j * G_CH
            pltpu.async_copy(
                p_hbm.at[ridx.at[pl.ds(off, G_CH)]], pbuf.at[b], gsem[b])
            pltpu.async_copy(
                q_hbm.at[cidx.at[pl.ds(off, G_CH)]], qbuf.at[b], gsem[b])

        for b in range(G_D):
            fire_gather(b, b)

        def group(g, _):
            for b in range(G_D):
                j = g * G_D + b
                pltpu.make_async_copy(
                    p_hbm.at[pl.ds(0, G_CH)], pbuf.at[b], gsem[b]).wait()
                pltpu.make_async_copy(
                    p_hbm.at[pl.ds(0, G_CH)], qbuf.at[b], gsem[b]).wait()

                @pl.when(g > 0)
                def _():
                    pltpu.make_async_copy(
                        sbuf.at[b], s_hbm.at[pl.ds(base0, G_CH)],
                        wsem[b]).wait()

                def rowi(i, _):
                    sbuf[b, i, :] = pbuf[b, i, :] + qbuf[b, i, :]
                    return 0
                lax.fori_loop(0, G_CH, rowi, 0, unroll=8)
                pltpu.async_copy(
                    sbuf.at[b], s_hbm.at[pl.ds(base0 + j * G_CH, G_CH)],
                    wsem[b])

                @pl.when(g < G_GRP - 1)
                def _():
                    fire_gather(j + G_D, b)
            return 0
        lax.fori_loop(0, G_GRP, group, 0)
        for b in range(G_D):
            pltpu.make_async_copy(
                sbuf.at[b], s_hbm.at[pl.ds(base0, G_CH)], wsem[b]).wait()

    return sc_gather


def _edge_body(s_ref, ea_ref, web_ref, w2b_ref, b1_ref, b2_ref, awb_ref,
               ab_ref, exp_ref, out_ref):
    pre = (s_ref[...]
           + jnp.dot(ea_ref[...], web_ref[...],
                     preferred_element_type=jnp.float32)
           + b1_ref[...])
    h = (jnp.dot(jax.nn.relu(pre), w2b_ref[...],
                 preferred_element_type=jnp.float32) + b2_ref[...])
    logit = (jnp.dot(h, awb_ref[...], preferred_element_type=jnp.float32)
             + ab_ref[...])
    a = jax.nn.sigmoid(logit)
    out_ref[...] = h * jnp.dot(a, exp_ref[...],
                               preferred_element_type=jnp.float32)


def _tc_edge(s8, ea8, web, w2b, b1t, b2t, awb, abt, exp8):
    er = E // 8
    br = 4000
    grid = er // br
    return pl.pallas_call(
        _edge_body,
        grid=(grid,),
        in_specs=[
            pl.BlockSpec((br, ND), lambda i: (i, 0)),
            pl.BlockSpec((br, ND), lambda i: (i, 0)),
            pl.BlockSpec((ND, ND), lambda i: (0, 0)),
            pl.BlockSpec((ND, ND), lambda i: (0, 0)),
            pl.BlockSpec((1, ND), lambda i: (0, 0)),
            pl.BlockSpec((1, ND), lambda i: (0, 0)),
            pl.BlockSpec((ND, NB), lambda i: (0, 0)),
            pl.BlockSpec((1, 1), lambda i: (0, 0)),
            pl.BlockSpec((NB, ND), lambda i: (0, 0)),
        ],
        out_specs=pl.BlockSpec((br, ND), lambda i: (i, 0)),
        out_shape=_sds((er, ND)),
    )(s8, ea8, web, w2b, b1t, b2t, awb, abt, exp8)


def _make_sc_scatter(mesh):
    @functools.partial(
        pl.kernel,
        out_type=(_sds((N, HD)), _sds((N, HD)), _sds((N,)), _sds((N,))),
        mesh=mesh,
        compiler_params=pltpu.CompilerParams(use_tc_tiling_on_sc=False),
        scratch_types=(
            [pltpu.VMEM((S_R, S_CH), jnp.int32),
             pltpu.VMEM((S_R, S_CH, HD), jnp.float32),
             pltpu.VMEM((S_CH,), jnp.float32),
             pltpu.VMEM((NPT, HD), jnp.float32),
             pltpu.VMEM((N,), jnp.float32),
             pltpu.VMEM_SHARED((N, HD), jnp.float32),
             pltpu.VMEM_SHARED((N,), jnp.float32)]
            + [pltpu.SemaphoreType.DMA] * (2 * S_R)
        ),
    )
    def sc_scatter(row_hbm, eout_hbm, sums0, sums1, cnt0, cnt1,
                   cidx, ebuf, ones, zbuf, zcnt, acc_sum, acc_cnt, *sems):
        lsem = sems[:S_R]
        ssem = sems[S_R:]
        c = lax.axis_index("c")
        s = lax.axis_index("s")
        wid = s * NC + c
        base0 = wid * EPW

        def zrow(i, _):
            zbuf[i, :] = jnp.zeros((HD,), jnp.float32)
            return 0
        lax.fori_loop(0, NPT, zrow, 0, unroll=4)

        def onesv(k, _):
            ones[pl.ds(k * HD, HD)] = jnp.ones((HD,), jnp.float32)
            return 0
        lax.fori_loop(0, S_CH // HD, onesv, 0)

        def zc(k, _):
            zcnt[pl.ds(k * HD, HD)] = jnp.zeros((HD,), jnp.float32)
            return 0
        lax.fori_loop(0, N // HD, zc, 0, unroll=4)

        pltpu.sync_copy(zbuf, acc_sum.at[pl.ds(s * NPT, NPT)])

        @pl.when(s == 0)
        def _():
            pltpu.sync_copy(zcnt, acc_cnt)

        plsc.subcore_barrier()

        def fire_loads(j, b):
            off = base0 + j * S_CH
            pltpu.async_copy(row_hbm.at[pl.ds(off, S_CH)], cidx.at[b],
                             lsem[b])
            pltpu.async_copy(eout_hbm.at[pl.ds(off, S_CH)], ebuf.at[b],
                             lsem[b])

        def drain_scatter(b):
            pltpu.make_async_copy(
                ebuf.at[b], acc_sum.at[cidx.at[b]], ssem[b]).wait()
            pltpu.make_async_copy(
                ones, acc_cnt.at[cidx.at[b]], ssem[b]).wait()

        for b in range(S_A):
            fire_loads(b, b)

        def group(g, _):
            for b in range(S_R):
                j = g * S_R + b
                pltpu.make_async_copy(
                    row_hbm.at[pl.ds(base0, S_CH)], cidx.at[b],
                    lsem[b]).wait()
                pltpu.make_async_copy(
                    eout_hbm.at[pl.ds(base0, S_CH)], ebuf.at[b],
                    lsem[b]).wait()
                pltpu.async_copy(ebuf.at[b], acc_sum.at[cidx.at[b]],
                                 ssem[b], add=True)
                pltpu.async_copy(ones, acc_cnt.at[cidx.at[b]],
                                 ssem[b], add=True)
                tb = (b + S_A) % S_R
                jn = j + S_A

                @pl.when(jn < S_NCH)
                def _():
                    @pl.when(j >= S_A)
                    def _():
                        drain_scatter(tb)
                    fire_loads(jn, tb)
            return 0
        lax.fori_loop(0, S_GRP, group, 0)
        for b in range(S_R):
            drain_scatter(b)

        plsc.subcore_barrier()

        @pl.when(c == 0)
        def _():
            pltpu.sync_copy(acc_sum.at[pl.ds(s * NPT, NPT)],
                            sums0.at[pl.ds(s * NPT, NPT)])

            @pl.when(s == 0)
            def _():
                pltpu.sync_copy(acc_cnt, cnt0)

        @pl.when(c == 1)
        def _():
            pltpu.sync_copy(acc_sum.at[pl.ds(s * NPT, NPT)],
                            sums1.at[pl.ds(s * NPT, NPT)])

            @pl.when(s == 0)
            def _():
                pltpu.sync_copy(acc_cnt, cnt1)

    return sc_scatter


_sc_cache = []


def _get_sc_kernels():
    if not _sc_cache:
        mesh = plsc.VectorSubcoreMesh(
            core_axis_name="c", subcore_axis_name="s",
            num_cores=NC, num_subcores=NS)
        _sc_cache.append((_make_sc_gather(mesh), _make_sc_scatter(mesh)))
    return _sc_cache[0]


def _node_body(x_ref, ps0_ref, ps1_ref, c0_ref, c1_ref, bat_ref,
               w1a_ref, w1b_ref, b1_ref, w12_ref, b12_ref, naw_ref, nab_ref,
               w2a_ref, w2b_ref, b2_ref, w22_ref, b22_ref,
               xo_ref, nms_ref, nmc_ref, ems_ref, emc_ref):
    i = pl.program_id(0)
    x = x_ref[...]
    psum = ps0_ref[...] + ps1_ref[...]
    craw = c0_ref[...] + c1_ref[...]
    agg = psum / jnp.maximum(craw, 1.0)
    pre1 = (jnp.dot(x, w1a_ref[...], preferred_element_type=jnp.float32)
            + jnp.dot(agg, w1b_ref[...], preferred_element_type=jnp.float32)
            + b1_ref[...])
    h1 = (jnp.dot(jax.nn.relu(pre1), w12_ref[...],
                  preferred_element_type=jnp.float32) + b12_ref[...])
    awg = jax.nn.sigmoid(
        jnp.dot(h1, naw_ref[...], preferred_element_type=jnp.float32)
        + nab_ref[...])
    h1 = h1 * awg
    pre2 = (jnp.dot(x, w2a_ref[...], preferred_element_type=jnp.float32)
            + jnp.dot(h1, w2b_ref[...], preferred_element_type=jnp.float32)
            + b2_ref[...])
    xo = (jnp.dot(jax.nn.relu(pre2), w22_ref[...],
                  preferred_element_type=jnp.float32) + b22_ref[...])
    xo_ref[...] = xo

    oh = (bat_ref[...] ==
          lax.broadcasted_iota(jnp.int32, (1, NB), 1).astype(jnp.float32)
          ).astype(jnp.float32)
    dn = (((0,), (0,)), ((), ()))

    @pl.when(i == 0)
    def _():
        nms_ref[...] = jnp.zeros_like(nms_ref)
        nmc_ref[...] = jnp.zeros_like(nmc_ref)
        ems_ref[...] = jnp.zeros_like(ems_ref)
        emc_ref[...] = jnp.zeros_like(emc_ref)

    nms_ref[...] += lax.dot_general(oh, xo, dn,
                                    preferred_element_type=jnp.float32)
    nmc_ref[...] += lax.dot_general(oh, jnp.ones_like(xo), dn,
                                    preferred_element_type=jnp.float32)
    ems_ref[...] += lax.dot_general(oh, psum, dn,
                                    preferred_element_type=jnp.float32)
    emc_ref[...] += lax.dot_general(
        oh, jnp.broadcast_to(craw, (craw.shape[0], HD)), dn,
        preferred_element_type=jnp.float32)


def _tc_node(x, ps0, ps1, c0, c1, batf,
             w1a, w1b, b1, w12, b12, naw, nab, w2a, w2b, b2, w22, b22):
    bn = 1000
    grid = N // bn
    return pl.pallas_call(
        _node_body,
        grid=(grid,),
        in_specs=[
            pl.BlockSpec((bn, ND), lambda i: (i, 0)),
            pl.BlockSpec((bn, HD), lambda i: (i, 0)),
            pl.BlockSpec((bn, HD), lambda i: (i, 0)),
            pl.BlockSpec((bn, 1), lambda i: (i, 0)),
            pl.BlockSpec((bn, 1), lambda i: (i, 0)),
            pl.BlockSpec((bn, 1), lambda i: (i, 0)),
            pl.BlockSpec((ND, HD), lambda i: (0, 0)),
            pl.BlockSpec((HD, HD), lambda i: (0, 0)),
            pl.BlockSpec((1, HD), lambda i: (0, 0)),
            pl.BlockSpec((HD, HD), lambda i: (0, 0)),
            pl.BlockSpec((1, HD), lambda i: (0, 0)),
            pl.BlockSpec((HD, 1), lambda i: (0, 0)),
            pl.BlockSpec((1, 1), lambda i: (0, 0)),
            pl.BlockSpec((ND, HD), lambda i: (0, 0)),
            pl.BlockSpec((HD, HD), lambda i: (0, 0)),
            pl.BlockSpec((1, HD), lambda i: (0, 0)),
            pl.BlockSpec((HD, ND), lambda i: (0, 0)),
            pl.BlockSpec((1, ND), lambda i: (0, 0)),
        ],
        out_specs=(
            pl.BlockSpec((bn, ND), lambda i: (i, 0)),
            pl.BlockSpec((NB, ND), lambda i: (0, 0)),
            pl.BlockSpec((NB, ND), lambda i: (0, 0)),
            pl.BlockSpec((NB, HD), lambda i: (0, 0)),
            pl.BlockSpec((NB, HD), lambda i: (0, 0)),
        ),
        out_shape=(
            _sds((N, ND)), _sds((NB, ND)), _sds((NB, ND)),
            _sds((NB, HD)), _sds((NB, HD)),
        ),
    )(x, ps0, ps1, c0, c1, batf,
      w1a, w1b, b1, w12, b12, naw, nab, w2a, w2b, b2, w22, b22)


def _global_body(u_ref, nms_ref, nmc_ref, ems_ref, emc_ref,
                 gu_ref, gn_ref, ge_ref, gb1_ref, gw2_ref, gb2_ref,
                 gaw_ref, gab_ref, uo_ref):
    nm = nms_ref[...] / jnp.maximum(nmc_ref[...], 1.0)
    em = ems_ref[...] / jnp.maximum(emc_ref[...], 1.0)
    pre = (jnp.dot(u_ref[...], gu_ref[...],
                   preferred_element_type=jnp.float32)
           + jnp.dot(nm, gn_ref[...], preferred_element_type=jnp.float32)
           + jnp.dot(em, ge_ref[...], preferred_element_type=jnp.float32)
           + gb1_ref[...])
    g = (jnp.dot(jax.nn.relu(pre), gw2_ref[...],
                 preferred_element_type=jnp.float32) + gb2_ref[...])
    ga = jax.nn.sigmoid(
        jnp.dot(g, gaw_ref[...], preferred_element_type=jnp.float32)
        + gab_ref[...])
    uo_ref[...] = g * ga


def _tc_global(u, nms, nmc, ems, emc, gu, gn, ge, gb1, gw2, gb2, gaw, gab):
    return pl.pallas_call(
        _global_body,
        out_shape=_sds((NB, GD)),
    )(u, nms, nmc, ems, emc, gu, gn, ge, gb1, gw2, gb2, gaw, gab)


def kernel(x, edge_index, edge_attr, u, batch,
           e_w1, e_b1, e_w2, e_b2, e_aw, e_ab,
           n1_w1, n1_b1, n1_w2, n1_b2, n_aw, n_ab,
           n2_w1, n2_b1, n2_w2, n2_b2,
           g_w1, g_b1, g_w2, g_b2, g_aw, g_ab):
    f32 = jnp.float32
    row = edge_index[0]
    col = edge_index[1]

    w_sd = jnp.concatenate([e_w1[:ND], e_w1[ND:2 * ND]], axis=1)
    we = e_w1[2 * ND:]
    eye8 = jnp.eye(NB, dtype=f32)
    web = jnp.kron(eye8, we)
    w2b = jnp.kron(eye8, e_w2)
    awb = jnp.kron(eye8, e_aw)
    exp8 = jnp.kron(eye8, jnp.ones((1, HD), f32))
    b1t = jnp.tile(e_b1, NB).reshape(1, ND)
    b2t = jnp.tile(e_b2, NB).reshape(1, ND)
    abt = e_ab.reshape(1, 1)

    sc_gather, sc_scatter = _get_sc_kernels()

    p, q = _tc_pq(x, w_sd)

    s = sc_gather(p, q, row, col)

    s8 = s.reshape(E // 8, ND)
    ea8 = edge_attr.reshape(E // 8, ND)
    eout8 = _tc_edge(s8, ea8, web, w2b, b1t, b2t, awb, abt, exp8)
    e_out = eout8.reshape(E, HD)

    sums0, sums1, cnt0, cnt1 = sc_scatter(row, e_out)

    batf = batch.astype(f32).reshape(N, 1)
    x_out, nms, nmc, ems, emc = _tc_node(
        x, sums0, sums1, cnt0.reshape(N, 1), cnt1.reshape(N, 1), batf,
        n1_w1[:ND], n1_w1[ND:], n1_b1.reshape(1, HD), n1_w2,
        n1_b2.reshape(1, HD), n_aw, n_ab.reshape(1, 1),
        n2_w1[:ND], n2_w1[ND:], n2_b1.reshape(1, HD), n2_w2,
        n2_b2.reshape(1, ND))

    u_out = _tc_global(
        u, nms, nmc, ems, emc,
        g_w1[:GD], g_w1[GD:GD + ND], g_w1[GD + ND:], g_b1.reshape(1, HD),
        g_w2, g_b2.reshape(1, GD), g_aw, g_ab.reshape(1, 1))

    return (x_out, e_out, u_out)
